# Initial kernel scaffold; baseline (speedup 1.0000x reference)
import jax
import jax.numpy as jnp
from jax.experimental import pallas as pl


def kernel(boxes, scores, regressions):
    raise NotImplementedError("write your pallas kernel here")


# speedup vs baseline: 17.9122x; 17.9122x over previous
import jax
import jax.numpy as jnp
from jax.experimental import pallas as pl

_B, _N, _C = 4, 20000, 80
_R, _L = 160, 128
_NP = _R * _L
_SCORE_TH = 0.05
_IOU_TH = 0.5
_PRE = 2000
_POST = 100
_IMG = 512.0
_OFF = 2.0 * _IMG


def _nms_kernel(sc_ref, bx_ref, rg_ref, bo_ref, so_ref, lo_ref):

    def cls_body(c, carry):
        best, besti = carry
        v = sc_ref[0, c]
        gt = v > best
        return jnp.where(gt, v, best), jnp.where(gt, c, besti)

    best0 = jnp.full((_R, _L), -jnp.inf, jnp.float32)
    besti0 = jnp.zeros((_R, _L), jnp.int32)
    best, labels = jax.lax.fori_loop(0, _C, cls_body, (best0, besti0))

    rowi = jax.lax.broadcasted_iota(jnp.int32, (_R, _L), 0)
    coli = jax.lax.broadcasted_iota(jnp.int32, (_R, _L), 1)
    gidx = rowi * _L + coli
    pad = gidx >= _N
    valid = best > _SCORE_TH
    s = jnp.where(pad, -2.0, jnp.where(valid, best, -1.0))

    bx1 = bx_ref[0, 0]
    by1 = bx_ref[0, 1]
    bx2 = bx_ref[0, 2]
    by2 = bx_ref[0, 3]
    dx = rg_ref[0, 0] * 0.1
    dy = rg_ref[0, 1] * 0.1
    dw = rg_ref[0, 2] * 0.2
    dh = rg_ref[0, 3] * 0.2
    w = bx2 - bx1
    h = by2 - by1
    cx = bx1 + 0.5 * w
    cy = by1 + 0.5 * h
    pcx = cx + dx * w
    pcy = cy + dy * h
    pw = w * jnp.exp(dw)
    ph = h * jnp.exp(dh)
    x1 = jnp.clip(pcx - 0.5 * pw, 0.0, _IMG)
    y1 = jnp.clip(pcy - 0.5 * ph, 0.0, _IMG)
    x2 = jnp.clip(pcx + 0.5 * pw, 0.0, _IMG)
    y2 = jnp.clip(pcy + 0.5 * ph, 0.0, _IMG)

    off = labels.astype(jnp.float32) * _OFF
    ox1 = x1 + off
    oy1 = y1 + off
    ox2 = x2 + off
    oy2 = y2 + off
    area = jnp.maximum(ox2 - ox1, 0.0) * jnp.maximum(oy2 - oy1, 0.0)

    bits = jax.lax.bitcast_convert_type(s, jnp.int32)
    key = jnp.where(bits >= 0, bits, -1 - (bits & jnp.int32(0x7FFFFFFF)))

    def bs_body(_, lohi):
        lo, hi = lohi
        mid = (lo >> 1) + (hi >> 1) + (lo & hi & 1)
        big = jnp.sum((key > mid).astype(jnp.int32)) >= _PRE
        return jnp.where(big, mid, lo), jnp.where(big, hi, mid)

    _, tau = jax.lax.fori_loop(
        0, 32, bs_body, (jnp.int32(-(2**31)), jnp.int32(2**31 - 1))
    )
    n1 = jnp.sum((key > tau).astype(jnp.int32))
    extra = _PRE - n1
    eq = key == tau

    def bs2_body(_, lohi):
        lo, hi = lohi
        mid = (lo + hi) >> 1
        geq = jnp.sum((eq & (gidx < mid)).astype(jnp.int32)) >= extra
        return jnp.where(geq, lo, mid), jnp.where(geq, mid, hi)

    _, mstar = jax.lax.fori_loop(
        0, 16, bs2_body, (jnp.int32(0), jnp.int32(_NP))
    )
    in2k = (key > tau) | (eq & (gidx < mstar))

    lane = jax.lax.broadcasted_iota(jnp.int32, (1, _L), 1)
    zrow = jnp.zeros((1, _L), jnp.float32)

    def sel_body(i, st):
        alive_i, avail_i, ob1, ob2, ob3, ob4, osc, olb = st
        alive = alive_i > 0
        avail = avail_i > 0
        keyv = jnp.where(alive, s, jnp.where(avail, s - 4.0, -1e9))
        mk = jnp.max(keyv)
        j = jnp.min(jnp.where(keyv == mk, gidx, _NP))
        one = gidx == j
        is1 = mk > 0.0
        z = jnp.float32(0.0)
        gx1 = jnp.sum(jnp.where(one, x1, z))
        gy1 = jnp.sum(jnp.where(one, y1, z))
        gx2 = jnp.sum(jnp.where(one, x2, z))
        gy2 = jnp.sum(jnp.where(one, y2, z))
        glb = jnp.sum(jnp.where(one, labels, 0))
        gsc = jnp.sum(jnp.where(one, s, z))
        ja = jnp.sum(jnp.where(one, area, z))
        offj = glb.astype(jnp.float32) * _OFF
        ix1 = jnp.maximum(ox1, gx1 + offj)
        iy1 = jnp.maximum(oy1, gy1 + offj)
        ix2 = jnp.minimum(ox2, gx2 + offj)
        iy2 = jnp.minimum(oy2, gy2 + offj)
        inter = jnp.maximum(ix2 - ix1, 0.0) * jnp.maximum(iy2 - iy1, 0.0)
        iou = inter / jnp.maximum(area + ja - inter, 1e-9)
        supp = iou > _IOU_TH
        alive_i = (alive & ~((supp & is1) | one)).astype(jnp.int32)
        avail_i = (avail & ~one).astype(jnp.int32)
        put = lane == i
        osc = jnp.where(put, jnp.where(is1, gsc, -1.0), osc)
        ob1 = jnp.where(put, gx1, ob1)
        ob2 = jnp.where(put, gy1, ob2)
        ob3 = jnp.where(put, gx2, ob3)
        ob4 = jnp.where(put, gy2, ob4)
        olb = jnp.where(put, glb, olb)
        return (alive_i, avail_i, ob1, ob2, ob3, ob4, osc, olb)

    alive0 = (in2k & (s > 0.0)).astype(jnp.int32)
    st = jax.lax.fori_loop(
        0,
        _POST,
        sel_body,
        (alive0, in2k.astype(jnp.int32), zrow, zrow, zrow, zrow, zrow,
         jnp.zeros((1, _L), jnp.int32)),
    )
    _, _, ob1, ob2, ob3, ob4, osc, olb = st
    bo_ref[0, 0:1, :] = ob1
    bo_ref[0, 1:2, :] = ob2
    bo_ref[0, 2:3, :] = ob3
    bo_ref[0, 3:4, :] = ob4
    bo_ref[0, 4:8, :] = jnp.zeros((4, _L), jnp.float32)
    so_ref[0, 0:1, :] = osc
    so_ref[0, 1:8, :] = jnp.zeros((7, _L), jnp.float32)
    lo_ref[0, 0:1, :] = olb
    lo_ref[0, 1:8, :] = jnp.zeros((7, _L), jnp.int32)


def _build(interpret=False):
    return pl.pallas_call(
        _nms_kernel,
        grid=(_B,),
        in_specs=[
            pl.BlockSpec((1, _C, _R, _L), lambda b: (b, 0, 0, 0)),
            pl.BlockSpec((1, 4, _R, _L), lambda b: (b, 0, 0, 0)),
            pl.BlockSpec((1, 4, _R, _L), lambda b: (b, 0, 0, 0)),
        ],
        out_specs=[
            pl.BlockSpec((1, 8, _L), lambda b: (b, 0, 0)),
            pl.BlockSpec((1, 8, _L), lambda b: (b, 0, 0)),
            pl.BlockSpec((1, 8, _L), lambda b: (b, 0, 0)),
        ],
        out_shape=[
            jax.ShapeDtypeStruct((_B, 8, _L), jnp.float32),
            jax.ShapeDtypeStruct((_B, 8, _L), jnp.float32),
            jax.ShapeDtypeStruct((_B, 8, _L), jnp.int32),
        ],
        interpret=interpret,
    )


def _prep(x):
    xt = jnp.transpose(x, (0, 2, 1))
    xt = jnp.pad(xt, ((0, 0), (0, 0), (0, _NP - _N)))
    return xt.reshape(_B, xt.shape[1], _R, _L)


@jax.jit
def _run(boxes, scores, regressions):
    bo, so, lo = _build()(_prep(scores), _prep(boxes), _prep(regressions))
    pred_boxes = jnp.transpose(bo[:, :4, :_POST], (0, 2, 1))
    return pred_boxes, so[:, 0, :_POST], lo[:, 0, :_POST]


def kernel(boxes, scores, regressions):
    return _run(boxes, scores, regressions)


# speedup vs baseline: 29.2450x; 1.6327x over previous
import jax
import jax.numpy as jnp
from jax.experimental import pallas as pl

_B, _N, _C = 4, 20000, 80
_R, _L = 160, 128
_RT = 16
_NP = _R * _L
_SCORE_TH = 0.05
_IOU_TH = 0.5
_PRE = 2000
_POST = 100
_IMG = 512.0
_OFF = 2.0 * _IMG


def _decode_kernel(sc_ref, bx_ref, rg_ref, s_ref, lb_ref, xy_ref):
    def cls_body(c, carry):
        best, besti = carry
        v = sc_ref[0, c]
        gt = v > best
        return jnp.where(gt, v, best), jnp.where(gt, c, besti)

    best0 = jnp.full((_RT, _L), -jnp.inf, jnp.float32)
    besti0 = jnp.zeros((_RT, _L), jnp.int32)
    best, labels = jax.lax.fori_loop(0, _C, cls_body, (best0, besti0))

    t = pl.program_id(1)
    rowi = jax.lax.broadcasted_iota(jnp.int32, (_RT, _L), 0) + t * _RT
    coli = jax.lax.broadcasted_iota(jnp.int32, (_RT, _L), 1)
    gidx = rowi * _L + coli
    pad = gidx >= _N
    valid = best > _SCORE_TH
    s_ref[0] = jnp.where(pad, -2.0, jnp.where(valid, best, -1.0))
    lb_ref[0] = labels

    bx1 = bx_ref[0, 0]
    by1 = bx_ref[0, 1]
    bx2 = bx_ref[0, 2]
    by2 = bx_ref[0, 3]
    dx = rg_ref[0, 0] * 0.1
    dy = rg_ref[0, 1] * 0.1
    dw = rg_ref[0, 2] * 0.2
    dh = rg_ref[0, 3] * 0.2
    w = bx2 - bx1
    h = by2 - by1
    cx = bx1 + 0.5 * w
    cy = by1 + 0.5 * h
    pcx = cx + dx * w
    pcy = cy + dy * h
    pw = w * jnp.exp(dw)
    ph = h * jnp.exp(dh)
    xy_ref[0, 0] = jnp.clip(pcx - 0.5 * pw, 0.0, _IMG)
    xy_ref[0, 1] = jnp.clip(pcy - 0.5 * ph, 0.0, _IMG)
    xy_ref[0, 2] = jnp.clip(pcx + 0.5 * pw, 0.0, _IMG)
    xy_ref[0, 3] = jnp.clip(pcy + 0.5 * ph, 0.0, _IMG)


def _rsum(x):
    return jnp.sum(jnp.sum(x, axis=1, keepdims=True), axis=2, keepdims=True)


def _rmax(x):
    return jnp.max(jnp.max(x, axis=1, keepdims=True), axis=2, keepdims=True)


def _rmin(x):
    return jnp.min(jnp.min(x, axis=1, keepdims=True), axis=2, keepdims=True)


def _nms_kernel(s_ref, lb_ref, xy_ref, bo_ref, so_ref, lo_ref):
    s = s_ref[...]
    labels = lb_ref[...]
    x1 = xy_ref[:, 0]
    y1 = xy_ref[:, 1]
    x2 = xy_ref[:, 2]
    y2 = xy_ref[:, 3]

    off = labels.astype(jnp.float32) * _OFF
    ox1 = x1 + off
    oy1 = y1 + off
    ox2 = x2 + off
    oy2 = y2 + off
    area = jnp.maximum(ox2 - ox1, 0.0) * jnp.maximum(oy2 - oy1, 0.0)

    rowi = jax.lax.broadcasted_iota(jnp.int32, (1, _R, _L), 1)
    coli = jax.lax.broadcasted_iota(jnp.int32, (1, _R, _L), 2)
    gidx = rowi * _L + coli

    bits = jax.lax.bitcast_convert_type(s, jnp.int32)
    key = jnp.where(bits >= 0, bits, -1 - (bits & jnp.int32(0x7FFFFFFF)))

    def bs_body(_, lohi):
        lo, hi = lohi
        mid = (lo >> 1) + (hi >> 1) + (lo & hi & 1)
        big = _rsum((key > mid).astype(jnp.int32)) >= _PRE
        return jnp.where(big, mid, lo), jnp.where(big, hi, mid)

    lo0 = jnp.full((_B, 1, 1), -(2**31), jnp.int32)
    hi0 = jnp.full((_B, 1, 1), 2**31 - 1, jnp.int32)
    _, tau = jax.lax.fori_loop(0, 32, bs_body, (lo0, hi0))
    n1 = _rsum((key > tau).astype(jnp.int32))
    extra = _PRE - n1
    eq = key == tau

    def bs2_body(_, lohi):
        lo, hi = lohi
        mid = (lo + hi) >> 1
        geq = _rsum((eq & (gidx < mid)).astype(jnp.int32)) >= extra
        return jnp.where(geq, lo, mid), jnp.where(geq, mid, hi)

    _, mstar = jax.lax.fori_loop(
        0, 16, bs2_body,
        (jnp.zeros((_B, 1, 1), jnp.int32), jnp.full((_B, 1, 1), _NP, jnp.int32)),
    )
    in2k = (key > tau) | (eq & (gidx < mstar))

    lane = jax.lax.broadcasted_iota(jnp.int32, (1, 1, _L), 2)
    zrow = jnp.zeros((_B, 1, _L), jnp.float32)

    def sel_body(i, st):
        alive_i, avail_i, ob1, ob2, ob3, ob4, osc, olb = st
        alive = alive_i > 0
        avail = avail_i > 0
        keyv = jnp.where(alive, s, jnp.where(avail, s - 4.0, -1e9))
        mk = _rmax(keyv)
        j = _rmin(jnp.where(keyv == mk, gidx, _NP))
        one = gidx == j
        is1 = mk > 0.0
        z = jnp.float32(0.0)
        gx1 = _rsum(jnp.where(one, x1, z))
        gy1 = _rsum(jnp.where(one, y1, z))
        gx2 = _rsum(jnp.where(one, x2, z))
        gy2 = _rsum(jnp.where(one, y2, z))
        glb = _rsum(jnp.where(one, labels, 0))
        gsc = _rsum(jnp.where(one, s, z))
        ja = _rsum(jnp.where(one, area, z))
        offj = glb.astype(jnp.float32) * _OFF
        ix1 = jnp.maximum(ox1, gx1 + offj)
        iy1 = jnp.maximum(oy1, gy1 + offj)
        ix2 = jnp.minimum(ox2, gx2 + offj)
        iy2 = jnp.minimum(oy2, gy2 + offj)
        inter = jnp.maximum(ix2 - ix1, 0.0) * jnp.maximum(iy2 - iy1, 0.0)
        iou = inter / jnp.maximum(area + ja - inter, 1e-9)
        supp = iou > _IOU_TH
        alive_i = (alive & ~((supp & is1) | one)).astype(jnp.int32)
        avail_i = (avail & ~one).astype(jnp.int32)
        put = lane == i
        osc = jnp.where(put, jnp.where(is1, gsc, -1.0), osc)
        ob1 = jnp.where(put, gx1, ob1)
        ob2 = jnp.where(put, gy1, ob2)
        ob3 = jnp.where(put, gx2, ob3)
        ob4 = jnp.where(put, gy2, ob4)
        olb = jnp.where(put, glb, olb)
        return (alive_i, avail_i, ob1, ob2, ob3, ob4, osc, olb)

    alive0 = (in2k & (s > 0.0)).astype(jnp.int32)
    st = jax.lax.fori_loop(
        0,
        _POST,
        sel_body,
        (alive0, in2k.astype(jnp.int32), zrow, zrow, zrow, zrow, zrow,
         jnp.zeros((_B, 1, _L), jnp.int32)),
    )
    _, _, ob1, ob2, ob3, ob4, osc, olb = st
    bo_ref[:, 0:1, :] = ob1
    bo_ref[:, 1:2, :] = ob2
    bo_ref[:, 2:3, :] = ob3
    bo_ref[:, 3:4, :] = ob4
    bo_ref[:, 4:8, :] = jnp.zeros((_B, 4, _L), jnp.float32)
    so_ref[:, 0:1, :] = osc
    so_ref[:, 1:8, :] = jnp.zeros((_B, 7, _L), jnp.float32)
    lo_ref[:, 0:1, :] = olb
    lo_ref[:, 1:8, :] = jnp.zeros((_B, 7, _L), jnp.int32)


def _build_decode(interpret=False):
    return pl.pallas_call(
        _decode_kernel,
        grid=(_B, _R // _RT),
        in_specs=[
            pl.BlockSpec((1, _C, _RT, _L), lambda b, t: (b, 0, t, 0)),
            pl.BlockSpec((1, 4, _RT, _L), lambda b, t: (b, 0, t, 0)),
            pl.BlockSpec((1, 4, _RT, _L), lambda b, t: (b, 0, t, 0)),
        ],
        out_specs=[
            pl.BlockSpec((1, _RT, _L), lambda b, t: (b, t, 0)),
            pl.BlockSpec((1, _RT, _L), lambda b, t: (b, t, 0)),
            pl.BlockSpec((1, 4, _RT, _L), lambda b, t: (b, 0, t, 0)),
        ],
        out_shape=[
            jax.ShapeDtypeStruct((_B, _R, _L), jnp.float32),
            jax.ShapeDtypeStruct((_B, _R, _L), jnp.int32),
            jax.ShapeDtypeStruct((_B, 4, _R, _L), jnp.float32),
        ],
        interpret=interpret,
    )


def _build_nms(interpret=False):
    return pl.pallas_call(
        _nms_kernel,
        in_specs=[
            pl.BlockSpec((_B, _R, _L), lambda: (0, 0, 0)),
            pl.BlockSpec((_B, _R, _L), lambda: (0, 0, 0)),
            pl.BlockSpec((_B, 4, _R, _L), lambda: (0, 0, 0, 0)),
        ],
        out_specs=[
            pl.BlockSpec((_B, 8, _L), lambda: (0, 0, 0)),
            pl.BlockSpec((_B, 8, _L), lambda: (0, 0, 0)),
            pl.BlockSpec((_B, 8, _L), lambda: (0, 0, 0)),
        ],
        out_shape=[
            jax.ShapeDtypeStruct((_B, 8, _L), jnp.float32),
            jax.ShapeDtypeStruct((_B, 8, _L), jnp.float32),
            jax.ShapeDtypeStruct((_B, 8, _L), jnp.int32),
        ],
        interpret=interpret,
    )


def _prep(x):
    xt = jnp.transpose(x, (0, 2, 1))
    xt = jnp.pad(xt, ((0, 0), (0, 0), (0, _NP - _N)))
    return xt.reshape(_B, xt.shape[1], _R, _L)


def _forward(boxes, scores, regressions, interpret=False):
    s, lb, xy = _build_decode(interpret)(
        _prep(scores), _prep(boxes), _prep(regressions)
    )
    bo, so, lo = _build_nms(interpret)(s, lb, xy)
    pred_boxes = jnp.transpose(bo[:, :4, :_POST], (0, 2, 1))
    return pred_boxes, so[:, 0, :_POST], lo[:, 0, :_POST]


@jax.jit
def _run(boxes, scores, regressions):
    return _forward(boxes, scores, regressions)


def kernel(boxes, scores, regressions):
    return _run(boxes, scores, regressions)
